# baseline (speedup 1.0000x reference)
import numpy as np
import jax
import jax.numpy as jnp
from jax.experimental import pallas as pl
from jax.experimental.pallas import tpu as pltpu

_N = 4096
_D = 64
_K = 16
_R = 512

_uniform_cache = []


def _uniform_noise():
    if not _uniform_cache:
        k0, k1 = np.uint32(0), np.uint32(42)
        ks2 = np.uint32(k0 ^ k1 ^ np.uint32(0x1BD11BDA))
        ks = [k0, k1, ks2]
        rots = [[13, 15, 26, 6], [17, 29, 16, 24]]

        def rot(v, d):
            return (v << np.uint32(d)) | (v >> np.uint32(32 - d))

        idx = np.arange(_N * _N, dtype=np.uint32)
        x0 = np.zeros_like(idx)
        x1 = idx
        with np.errstate(over="ignore"):
            x0 = x0 + k0
            x1 = x1 + k1
            for i in range(5):
                for r in rots[i % 2]:
                    x0 = x0 + x1
                    x1 = rot(x1, r)
                    x1 = x1 ^ x0
                x0 = x0 + ks[(i + 1) % 3]
                x1 = x1 + ks[(i + 2) % 3] + np.uint32(i + 1)
        bits = x0 ^ x1
        fl = ((bits >> np.uint32(9)) | np.uint32(0x3F800000)).view(np.float32)
        fl = fl - np.float32(1.0)
        u = fl * (np.float32(1.0) - np.float32(1e-8)) + np.float32(1e-8)
        u = np.maximum(np.float32(1e-8), u)
        _uniform_cache.append(u.reshape(_N, _N))
    return _uniform_cache[0]


def _project(x):
    nrm = jnp.sqrt(jnp.sum(x * x, axis=1, keepdims=True))
    scale = (jnp.maximum(nrm - 1.0, 0.0) + 1.0) * (1.0 + 1e-2)
    xh = x / scale
    return xh, jnp.sum(xh * xh, axis=1)


def _edge_kernel(x_ref, xb_ref, g_ref, t_ref, idx_ref, w_ref, xh_scr, sq_scr):
    i = pl.program_id(0)

    @pl.when(i == 0)
    def _():
        xh0, sq0 = _project(x_ref[...])
        xh_scr[...] = xh0
        sq_scr[...] = sq0[None, :]

    xh = xh_scr[...]
    sq = sq_scr[...]
    xb, sqb = _project(xb_ref[...])
    pq = sqb[:, None] + sq - 2.0 * jax.lax.dot_general(
        xb, xh, (((1,), (1,)), ((), ())), preferred_element_type=jnp.float32)
    pq = jnp.maximum(pq, 0.0)
    arg = 1e-6 + 1.0 + 2.0 * pq / ((1.0 - sqb)[:, None] * (1.0 - sq))
    acosh = jnp.log(arg + jnp.sqrt((arg - 1.0) * (arg + 1.0)))
    dist = acosh * acosh
    t = jnp.clip(t_ref[0, 0], 0.0, 5.0)
    g = -jnp.log(-jnp.log(g_ref[...]))
    z = (-dist * jnp.exp(t) + g) / t
    m = jnp.max(z, axis=1, keepdims=True)
    e = jnp.exp(z - m)
    s = jnp.sum(e, axis=1, keepdims=True)
    p = e / s
    colidx = jax.lax.broadcasted_iota(jnp.int32, (_R, _N), 1)
    vals, idxs = [], []
    for _ in range(_K):
        mv = jnp.max(p, axis=1, keepdims=True)
        ji = jnp.min(jnp.where(p == mv, colidx, _N), axis=1, keepdims=True)
        vals.append(mv)
        idxs.append(ji)
        p = jnp.where(colidx == ji, -jnp.inf, p)
    idx_ref[...] = jnp.concatenate(idxs, axis=1)
    w_ref[...] = jnp.concatenate(vals, axis=1)


def kernel(x, temperature):
    g = _uniform_noise()
    t2 = jnp.reshape(temperature.astype(jnp.float32), (1, 1))
    idx, w = pl.pallas_call(
        _edge_kernel,
        grid=(_N // _R,),
        in_specs=[
            pl.BlockSpec((_N, _D), lambda i: (0, 0)),
            pl.BlockSpec((_R, _D), lambda i: (i, 0)),
            pl.BlockSpec((_R, _N), lambda i: (i, 0)),
            pl.BlockSpec((1, 1), lambda i: (0, 0)),
        ],
        out_specs=[
            pl.BlockSpec((_R, _K), lambda i: (i, 0)),
            pl.BlockSpec((_R, _K), lambda i: (i, 0)),
        ],
        out_shape=[
            jax.ShapeDtypeStruct((_N, _K), jnp.int32),
            jax.ShapeDtypeStruct((_N, _K), jnp.float32),
        ],
        scratch_shapes=[
            pltpu.VMEM((_N, _D), jnp.float32),
            pltpu.VMEM((1, _N), jnp.float32),
        ],
    )(x, x, g, t2)
    rows = jax.lax.broadcasted_iota(jnp.int32, (_N, _K), 0)
    edges = jnp.stack((rows.reshape(-1), idx.reshape(-1)), axis=0)
    return (x, edges, w.reshape(-1))


# speedup vs baseline: 1.2842x; 1.2842x over previous
import numpy as np
import jax
import jax.numpy as jnp
from jax.experimental import pallas as pl
from jax.experimental.pallas import tpu as pltpu

_N = 4096
_D = 64
_K = 16
_R = 256

_uniform_cache = []


def _uniform_noise():
    if not _uniform_cache:
        k0, k1 = np.uint32(0), np.uint32(42)
        ks2 = np.uint32(k0 ^ k1 ^ np.uint32(0x1BD11BDA))
        ks = [k0, k1, ks2]
        rots = [[13, 15, 26, 6], [17, 29, 16, 24]]

        def rot(v, d):
            return (v << np.uint32(d)) | (v >> np.uint32(32 - d))

        idx = np.arange(_N * _N, dtype=np.uint32)
        x0 = np.zeros_like(idx)
        x1 = idx
        with np.errstate(over="ignore"):
            x0 = x0 + k0
            x1 = x1 + k1
            for i in range(5):
                for r in rots[i % 2]:
                    x0 = x0 + x1
                    x1 = rot(x1, r)
                    x1 = x1 ^ x0
                x0 = x0 + ks[(i + 1) % 3]
                x1 = x1 + ks[(i + 2) % 3] + np.uint32(i + 1)
        bits = x0 ^ x1
        fl = ((bits >> np.uint32(9)) | np.uint32(0x3F800000)).view(np.float32)
        fl = fl - np.float32(1.0)
        u = fl * (np.float32(1.0) - np.float32(1e-8)) + np.float32(1e-8)
        u = np.maximum(np.float32(1e-8), u)
        g = (-np.log(-np.log(u.astype(np.float64)))).astype(np.float32)
        _uniform_cache.append(g.reshape(_N, _N))
    return _uniform_cache[0]


def _project(x):
    nrm = jnp.sqrt(jnp.sum(x * x, axis=1, keepdims=True))
    scale = (jnp.maximum(nrm - 1.0, 0.0) + 1.0) * (1.0 + 1e-2)
    xh = x / scale
    return xh, jnp.sum(xh * xh, axis=1)


def _edge_kernel(x_ref, xb_ref, g_ref, t_ref, idx_ref, w_ref, xh_scr, sq_scr):
    i = pl.program_id(0)

    @pl.when(i == 0)
    def _():
        xh0, sq0 = _project(x_ref[...])
        xh_scr[...] = xh0
        sq_scr[...] = sq0[None, :]

    xh = xh_scr[...]
    sq = sq_scr[...]
    xb, sqb = _project(xb_ref[...])
    pq = sqb[:, None] + sq - 2.0 * jax.lax.dot_general(
        xb, xh, (((1,), (1,)), ((), ())), preferred_element_type=jnp.float32)
    pq = jnp.maximum(pq, 0.0)
    arg = 1e-6 + 1.0 + 2.0 * pq / ((1.0 - sqb)[:, None] * (1.0 - sq))
    acosh = jnp.log(arg + jnp.sqrt((arg - 1.0) * (arg + 1.0)))
    dist = acosh * acosh
    t = jnp.clip(t_ref[0, 0], 0.0, 5.0)
    z = (-dist * jnp.exp(t) + g_ref[...]) / t
    m = jnp.max(z, axis=1, keepdims=True)
    e = jnp.exp(z - m)
    s = jnp.sum(e, axis=1, keepdims=True)
    p = e / s
    colidx = jax.lax.broadcasted_iota(jnp.int32, (_R, _N), 1)
    vals, idxs = [], []
    for _ in range(_K):
        mv = jnp.max(p, axis=1, keepdims=True)
        ji = jnp.min(jnp.where(p == mv, colidx, _N), axis=1, keepdims=True)
        vals.append(mv)
        idxs.append(ji)
        p = jnp.where(colidx == ji, -jnp.inf, p)
    idx_ref[...] = jnp.concatenate(idxs, axis=1)
    w_ref[...] = jnp.concatenate(vals, axis=1)


def kernel(x, temperature):
    g = _uniform_noise()
    t2 = jnp.reshape(temperature.astype(jnp.float32), (1, 1))
    idx, w = pl.pallas_call(
        _edge_kernel,
        grid=(_N // _R,),
        in_specs=[
            pl.BlockSpec((_N, _D), lambda i: (0, 0)),
            pl.BlockSpec((_R, _D), lambda i: (i, 0)),
            pl.BlockSpec((_R, _N), lambda i: (i, 0)),
            pl.BlockSpec((1, 1), lambda i: (0, 0)),
        ],
        out_specs=[
            pl.BlockSpec((_R, _K), lambda i: (i, 0)),
            pl.BlockSpec((_R, _K), lambda i: (i, 0)),
        ],
        out_shape=[
            jax.ShapeDtypeStruct((_N, _K), jnp.int32),
            jax.ShapeDtypeStruct((_N, _K), jnp.float32),
        ],
        scratch_shapes=[
            pltpu.VMEM((_N, _D), jnp.float32),
            pltpu.VMEM((1, _N), jnp.float32),
        ],
    )(x, x, g, t2)
    rows = jax.lax.broadcasted_iota(jnp.int32, (_N, _K), 0)
    edges = jnp.stack((rows.reshape(-1), idx.reshape(-1)), axis=0)
    return (x, edges, w.reshape(-1))
